# baseline (speedup 1.0000x reference)
import functools

import jax
import jax.numpy as jnp
from jax import lax
from jax.experimental import pallas as pl
from jax.experimental.pallas import tpu as pltpu
from jax.experimental.pallas import tpu_sc as plsc

N_NODES = 100000
N_FEATS = 9
EMB_DIM = 512
NUM_EMB = 119

_INFO = plsc.get_sparse_core_info()
NC = _INFO.num_cores
NS = _INFO.num_subcores
L = _INFO.num_lanes
NW = NC * NS
N_REP = NW

WINDOW = 3200
CHUNK = 32
N_CHUNKS = WINDOW // CHUNK
SUPER = 5
N_SUPER = N_CHUNKS // SUPER
N_EXTRACT = WINDOW // L
DGRP = EMB_DIM // L
HALF = L
REP_STRIDE = 120


def _sc_body(x_hbm, table_hbm, out_hbm, idx_v, table_v, idx_sm,
             gsems, csems):
    wid = lax.axis_index("s") * NC + lax.axis_index("c")
    start = jnp.minimum(wid * WINDOW, N_NODES - WINDOW)
    rep_off = wid * REP_STRIDE

    pltpu.sync_copy(
        table_hbm.at[pl.ds(pl.multiple_of(rep_off, 8), REP_STRIDE)], table_v)

    def phase1(xwin_v):
        pltpu.sync_copy(
            x_hbm.at[pl.ds(start * N_FEATS, WINDOW * N_FEATS)], xwin_v)

        def extract(j, _):
            flat16 = (j * L + lax.iota(jnp.int32, L)) * N_FEATS
            idx_v[pl.ds(j * L, L)] = (
                plsc.load_gather(xwin_v, [flat16]) + rep_off
            )
            return _

        lax.fori_loop(0, N_EXTRACT, extract, None)

    pl.run_scoped(phase1, pltpu.VMEM((WINDOW * N_FEATS,), jnp.int32))

    def phase2(g0, g1, g2, c0, c1):
        gbufs = (g0, g1, g2)
        cbufs = (c0, c1)

        def drain(buf, sem, rows):
            pltpu.make_async_copy(buf, out_hbm.at[pl.ds(start, rows)],
                                  sem).wait()

        def construct(cid, half, cb, cs):
            raw = idx_v[pl.ds(cid * CHUNK + half * HALF, L)] - rep_off
            for i in range(L):
                idx_sm[i] = raw[i]

            @plsc.parallel_loop(0, HALF, unroll=2)
            def row(i):
                rr = idx_sm[i]
                for w in range(DGRP // 8):
                    vals = [table_v[rr, pl.ds((w * 8 + j) * L, L)]
                            for j in range(8)]
                    for j in range(8):
                        cb[i, pl.ds((w * 8 + j) * L, L)] = vals[j]

            pltpu.async_copy(
                cb,
                out_hbm.at[pl.ds(start + cid * CHUNK + half * HALF, HALF)],
                cs)

        def superstep(s, _):
            base_cid = s * SUPER

            for gi in range(3):
                cid = base_cid + gi
                gbuf, gsem = gbufs[gi], gsems.at[gi]

                @pl.when(s >= 1)
                def _(gbuf=gbuf, gsem=gsem):
                    drain(gbuf, gsem, CHUNK)

                pltpu.async_copy(
                    table_hbm.at[idx_v.at[pl.ds(cid * CHUNK, CHUNK)]],
                    gbuf, gsem)

            for t in range(4):
                cid = base_cid + 3 + t // 2
                cb, cs = cbufs[t % 2], csems.at[t % 2]
                if t < 2:
                    @pl.when(s >= 1)
                    def _(cb=cb, cs=cs):
                        drain(cb, cs, HALF)
                else:
                    drain(cb, cs, HALF)
                construct(cid, t % 2, cb, cs)

            for gi in range(3):
                cid = base_cid + gi
                gbuf, gsem = gbufs[gi], gsems.at[gi]
                pltpu.make_async_copy(
                    table_hbm.at[idx_v.at[pl.ds(cid * CHUNK, CHUNK)]],
                    gbuf, gsem).wait()
                pltpu.async_copy(
                    gbuf, out_hbm.at[pl.ds(start + cid * CHUNK, CHUNK)],
                    gsem)

            return _

        lax.fori_loop(0, N_SUPER, superstep, None)

        for gi in range(3):
            drain(gbufs[gi], gsems.at[gi], CHUNK)
        for t in range(2):
            drain(cbufs[t], csems.at[t], HALF)

    pl.run_scoped(
        phase2,
        pltpu.VMEM((CHUNK, EMB_DIM), jnp.float32),
        pltpu.VMEM((CHUNK, EMB_DIM), jnp.float32),
        pltpu.VMEM((CHUNK, EMB_DIM), jnp.float32),
        pltpu.VMEM((HALF, EMB_DIM), jnp.float32),
        pltpu.VMEM((HALF, EMB_DIM), jnp.float32),
    )


@jax.jit
def kernel(x, atom_embedding_weight):
    mesh = plsc.VectorSubcoreMesh(core_axis_name="c", subcore_axis_name="s")
    run = functools.partial(
        pl.kernel,
        mesh=mesh,
        out_type=jax.ShapeDtypeStruct((N_NODES, EMB_DIM), jnp.float32),
        scratch_types=[
            pltpu.VMEM((WINDOW,), jnp.int32),
            pltpu.VMEM((REP_STRIDE, EMB_DIM), jnp.float32),
            pltpu.SMEM((L,), jnp.int32),
            pltpu.SemaphoreType.DMA((3,)),
            pltpu.SemaphoreType.DMA((2,)),
        ],
        compiler_params=pltpu.CompilerParams(needs_layout_passes=False),
    )(_sc_body)
    table_rep = jnp.tile(
        jnp.pad(atom_embedding_weight, ((0, REP_STRIDE - NUM_EMB), (0, 0))),
        (N_REP, 1))
    out = run(x.reshape(-1), table_rep)
    return out


# speedup vs baseline: 1.0353x; 1.0353x over previous
import functools

import jax
import jax.numpy as jnp
from jax import lax
from jax.experimental import pallas as pl
from jax.experimental.pallas import tpu as pltpu
from jax.experimental.pallas import tpu_sc as plsc

N_NODES = 100000
N_FEATS = 9
EMB_DIM = 512
NUM_EMB = 119

_INFO = plsc.get_sparse_core_info()
NC = _INFO.num_cores
NS = _INFO.num_subcores
L = _INFO.num_lanes
NW = NC * NS
N_REP = NW

WINDOW = 3200
CHUNK = 32
N_CHUNKS = WINDOW // CHUNK
SUPER = 4
N_GATHER = 2
N_SUPER = N_CHUNKS // SUPER
N_EXTRACT = WINDOW // L
DGRP = EMB_DIM // L
HALF = L
REP_STRIDE = 120


def _sc_body(x_hbm, table_hbm, out_hbm, idx_v, table_v, idx_sm,
             gsems, csems):
    wid = lax.axis_index("s") * NC + lax.axis_index("c")
    start = jnp.minimum(wid * WINDOW, N_NODES - WINDOW)
    rep_off = wid * REP_STRIDE

    pltpu.sync_copy(
        table_hbm.at[pl.ds(pl.multiple_of(rep_off, 8), REP_STRIDE)], table_v)

    def phase1(xwin_v):
        pltpu.sync_copy(
            x_hbm.at[pl.ds(start * N_FEATS, WINDOW * N_FEATS)], xwin_v)

        def extract(j, _):
            flat16 = (j * L + lax.iota(jnp.int32, L)) * N_FEATS
            idx_v[pl.ds(j * L, L)] = (
                plsc.load_gather(xwin_v, [flat16]) + rep_off
            )
            return _

        lax.fori_loop(0, N_EXTRACT, extract, None)

    pl.run_scoped(phase1, pltpu.VMEM((WINDOW * N_FEATS,), jnp.int32))

    def phase2(g0, g1, c0, c1):
        gbufs = (g0, g1)
        cbufs = (c0, c1)

        def drain(buf, sem, rows):
            pltpu.make_async_copy(buf, out_hbm.at[pl.ds(start, rows)],
                                  sem).wait()

        def construct(cid, half, cb, cs):
            raw = idx_v[pl.ds(cid * CHUNK + half * HALF, L)] - rep_off
            for i in range(L):
                idx_sm[i] = raw[i]

            @plsc.parallel_loop(0, HALF, unroll=2)
            def row(i):
                rr = idx_sm[i]
                for w in range(DGRP // 8):
                    vals = [table_v[rr, pl.ds((w * 8 + j) * L, L)]
                            for j in range(8)]
                    for j in range(8):
                        cb[i, pl.ds((w * 8 + j) * L, L)] = vals[j]

            pltpu.async_copy(
                cb,
                out_hbm.at[pl.ds(start + cid * CHUNK + half * HALF, HALF)],
                cs)

        def superstep(s, _):
            base_cid = s * SUPER

            for gi in range(N_GATHER):
                cid = base_cid + gi
                gbuf, gsem = gbufs[gi], gsems.at[gi]

                @pl.when(s >= 1)
                def _(gbuf=gbuf, gsem=gsem):
                    drain(gbuf, gsem, CHUNK)

                pltpu.async_copy(
                    table_hbm.at[idx_v.at[pl.ds(cid * CHUNK, CHUNK)]],
                    gbuf, gsem)

            for t in range(4):
                cid = base_cid + N_GATHER + t // 2
                cb, cs = cbufs[t % 2], csems.at[t % 2]
                if t < 2:
                    @pl.when(s >= 1)
                    def _(cb=cb, cs=cs):
                        drain(cb, cs, HALF)
                else:
                    drain(cb, cs, HALF)
                construct(cid, t % 2, cb, cs)

            for gi in range(N_GATHER):
                cid = base_cid + gi
                gbuf, gsem = gbufs[gi], gsems.at[gi]
                pltpu.make_async_copy(
                    table_hbm.at[idx_v.at[pl.ds(cid * CHUNK, CHUNK)]],
                    gbuf, gsem).wait()
                pltpu.async_copy(
                    gbuf, out_hbm.at[pl.ds(start + cid * CHUNK, CHUNK)],
                    gsem)

            return _

        lax.fori_loop(0, N_SUPER, superstep, None)

        for gi in range(N_GATHER):
            drain(gbufs[gi], gsems.at[gi], CHUNK)
        for t in range(2):
            drain(cbufs[t], csems.at[t], HALF)

    pl.run_scoped(
        phase2,
        pltpu.VMEM((CHUNK, EMB_DIM), jnp.float32),
        pltpu.VMEM((CHUNK, EMB_DIM), jnp.float32),
        pltpu.VMEM((HALF, EMB_DIM), jnp.float32),
        pltpu.VMEM((HALF, EMB_DIM), jnp.float32),
    )


@jax.jit
def kernel(x, atom_embedding_weight):
    mesh = plsc.VectorSubcoreMesh(core_axis_name="c", subcore_axis_name="s")
    run = functools.partial(
        pl.kernel,
        mesh=mesh,
        out_type=jax.ShapeDtypeStruct((N_NODES, EMB_DIM), jnp.float32),
        scratch_types=[
            pltpu.VMEM((WINDOW,), jnp.int32),
            pltpu.VMEM((REP_STRIDE, EMB_DIM), jnp.float32),
            pltpu.SMEM((L,), jnp.int32),
            pltpu.SemaphoreType.DMA((N_GATHER,)),
            pltpu.SemaphoreType.DMA((2,)),
        ],
        compiler_params=pltpu.CompilerParams(needs_layout_passes=False),
    )(_sc_body)
    table_rep = jnp.tile(
        jnp.pad(atom_embedding_weight, ((0, REP_STRIDE - NUM_EMB), (0, 0))),
        (N_REP, 1))
    out = run(x.reshape(-1), table_rep)
    return out
